# baseline (speedup 1.0000x reference)
import functools

import jax
import jax.numpy as jnp
from jax import lax
from jax.experimental import pallas as pl
from jax.experimental.pallas import tpu as pltpu
from jax.experimental.pallas import tpu_sc as plsc

E = 8
B = 32
D = 1024
H1, H2, H3 = 512, 256, 128
V = 50257

NC = 2
NS = 16
L = 16

C = 1024
NCH = 50
VP = C * NCH
OC = 2048
ONCH = VP // OC
OUT_W = (V + 1) * 2


def _routing_body(x_ref, w1, b1, g1, be1, w2, b2, g2, be2, w3, b3, g3, be3,
                  wo, bo, out_ref):
    eps = 1e-6
    x = x_ref[...]
    for (w, b, g, be, n) in ((w1, b1, g1, be1, H1),
                             (w2, b2, g2, be2, H2),
                             (w3, b3, g3, be3, H3)):
        h = lax.dot_general(x, w[...], (((1,), (1,)), ((), ())),
                            preferred_element_type=jnp.float32)
        h = h + b[...]
        h = jnp.where(h > 0, h, jnp.exp(jnp.minimum(h, 0.0)) - 1.0)
        mean = jnp.mean(h, axis=-1, keepdims=True)
        var = jnp.sum((h - mean) ** 2, axis=-1, keepdims=True) / (n - 1)
        x = g[...] * (h - mean) / (jnp.sqrt(var) + eps) + be[...]
    logits = lax.dot_general(x, wo[...], (((1,), (1,)), ((), ())),
                             preferred_element_type=jnp.float32)
    logits = logits.reshape(E, B)
    m = jnp.max(logits, axis=-1, keepdims=True)
    ex = jnp.exp(logits - m)
    out_ref[...] = ex / jnp.sum(ex, axis=-1, keepdims=True)


def _routing(x, w1, b1, g1, be1, w2, b2, g2, be2, w3, b3, g3, be3, wo, bo):
    return pl.pallas_call(
        _routing_body,
        out_shape=jax.ShapeDtypeStruct((E, B), jnp.float32),
    )(x, w1, b1, g1, be1, w2, b2, g2, be2, w3, b3, g3, be3, wo, bo)


def _mix_body(pr_hbm, ids_hbm, w_hbm, out_hbm,
              pr_buf, ids_buf, w_buf, resp, acc, ostage, sems):
    b = lax.axis_index("s") * NC + lax.axis_index("c")
    iota = lax.iota(jnp.int32, L)
    zero = jnp.zeros((L,), jnp.float32)

    def _issue(row, c, slot):
        pltpu.async_copy(pr_hbm.at[row, pl.ds(c * C, C)],
                         pr_buf.at[pl.ds(slot * C, C)], sems.at[slot, 0])
        pltpu.async_copy(ids_hbm.at[row, pl.ds(c * C, C)],
                         ids_buf.at[pl.ds(slot * C, C)], sems.at[slot, 1])

    def _wait(row, c, slot):
        pltpu.make_async_copy(pr_hbm.at[row, pl.ds(c * C, C)],
                              pr_buf.at[pl.ds(slot * C, C)],
                              sems.at[slot, 0]).wait()
        pltpu.make_async_copy(ids_hbm.at[row, pl.ds(c * C, C)],
                              ids_buf.at[pl.ds(slot * C, C)],
                              sems.at[slot, 1]).wait()

    pltpu.sync_copy(w_hbm.at[b], w_buf)
    _issue(b, 0, 0)

    def _clear(i, _):
        s = pl.ds(pl.multiple_of(i * L, L), L)
        resp[s] = zero
        acc[s] = zero
        return 0

    lax.fori_loop(0, VP // L, _clear, 0)

    for e in range(E):
        w_vec = w_buf[e]
        row = e * B + b

        def _pair(g, _, w_vec=w_vec, row=row):
            for slot in (0, 1):
                c = g * 2 + slot
                _wait(row, c, slot)

                @pl.when(c + 1 < NCH)
                def _():
                    _issue(row, c + 1, 1 - slot)

                def _rmw(i, pend, slot=slot, w_vec=w_vec):
                    s = pl.ds(pl.multiple_of(slot * C + i * L, L), L)
                    p = pr_buf[s] * w_vec
                    t = ids_buf[s]
                    cur = plsc.load_gather(resp, [t])
                    m = jnp.maximum(cur, p)
                    plsc.store_scatter(resp, [t], m)
                    back = plsc.load_gather(resp, [t])
                    return pend | (m > back)

                nofix = jnp.zeros((L,), jnp.bool_)
                pend = lax.fori_loop(0, C // L, _rmw, nofix)
                lax.while_loop(
                    jnp.any,
                    lambda _, _rmw=_rmw, nofix=nofix: lax.fori_loop(
                        0, C // L, _rmw, nofix),
                    pend)
            return 0

        lax.fori_loop(0, NCH // 2, _pair, 0)
        if e + 1 < E:
            _issue((e + 1) * B + b, 0, 0)

        def _accum(i, _):
            s = pl.ds(pl.multiple_of(i * L, L), L)
            acc[s] = acc[s] + resp[s]
            resp[s] = zero
            return 0

        lax.fori_loop(0, VP // L, _accum, 0)

    even = iota * 2
    odd = even + 1

    def _ochunk(c, _):
        def _ov(i, _):
            off = c * OC + i * L
            s = pl.ds(pl.multiple_of(off, L), L)
            p = acc[s]
            vid = off + iota
            vf = jnp.where(vid == V, -1.0, vid.astype(jnp.float32))
            base = pl.multiple_of(i * L * 2, 2 * L)
            plsc.store_scatter(ostage, [base + even], p)
            plsc.store_scatter(ostage, [base + odd], vf)
            return 0

        lax.fori_loop(0, OC // L, _ov, 0)
        return 0

    def _owrite(c, _):
        _ochunk(c, 0)
        pltpu.sync_copy(ostage, out_hbm.at[b, pl.ds(c * 2 * OC, 2 * OC)])
        return 0

    lax.fori_loop(0, ONCH, _owrite, 0)


@functools.partial(jax.jit, static_argnames=())
def _mix(pr, ids, wb):
    mesh = plsc.VectorSubcoreMesh(core_axis_name="c", subcore_axis_name="s",
                                  num_cores=NC, num_subcores=NS)
    f = pl.kernel(
        _mix_body,
        out_type=jax.ShapeDtypeStruct((B, 2 * VP), jnp.float32),
        mesh=mesh,
        compiler_params=pltpu.CompilerParams(needs_layout_passes=False),
        scratch_types=[
            pltpu.VMEM((2 * C,), jnp.float32),
            pltpu.VMEM((2 * C,), jnp.int32),
            pltpu.VMEM((E, L), jnp.float32),
            pltpu.VMEM((VP,), jnp.float32),
            pltpu.VMEM((VP,), jnp.float32),
            pltpu.VMEM((2 * OC,), jnp.float32),
            pltpu.SemaphoreType.DMA((2, 2)),
        ],
    )
    return f(pr, ids, wb)


def kernel(endpoint_emb, prediction, W1, b1, g1, be1, W2, b2, g2, be2,
           W3, b3, g3, be3, Wo, bo):
    x = endpoint_emb.reshape(E * B, D)
    routing = _routing(
        x, W1, b1.reshape(1, H1), g1.reshape(1, H1), be1.reshape(1, H1),
        W2, b2.reshape(1, H2), g2.reshape(1, H2), be2.reshape(1, H2),
        W3, b3.reshape(1, H3), g3.reshape(1, H3), be3.reshape(1, H3),
        Wo, bo.reshape(1, 1))
    wb = jnp.broadcast_to(routing.T[:, :, None], (B, E, L))
    pr = prediction[:, :, :V, 0]
    ids = prediction[:, :, :V, 1].astype(jnp.int32)
    pr_p = jnp.pad(pr, ((0, 0), (0, 0), (0, VP - V))).reshape(E * B, VP)
    ids_p = jnp.pad(ids, ((0, 0), (0, 0), (0, VP - V))).reshape(E * B, VP)
    out = _mix(pr_p, ids_p, wb)
    return out[:, :OUT_W].reshape(B, V + 1, 2)
